# Initial kernel scaffold; baseline (speedup 1.0000x reference)
import jax
import jax.numpy as jnp
from jax.experimental import pallas as pl


def kernel(x, edge_index, edge_attr, node_to_subgraph, subgraph_to_graph, nn_w1, nn_b1, nn_w2, nn_b2, root_w, conv_b, fc1_w, fc1_b, fc2_w, fc2_b, fc3_w, fc3_b):
    raise NotImplementedError("write your pallas kernel here")


# speedup vs baseline: 1.6216x; 1.6216x over previous
import functools

import jax
import jax.numpy as jnp
from jax import lax
from jax.experimental import pallas as pl
from jax.experimental.pallas import tpu as pltpu
from jax.experimental.pallas import tpu_sc as plsc

N = 10000
NP = 10240
E = 160000
EP = 163840
ET = 2048
NSUB = 2000
NSUBP = 2048
NGRAPH = 64
D_INT = 16
M_OUT = 32
HID = 128

NC = 2
NS = 16
NW = NC * NS
EPW = EP // NW
CH = 1024
NCHUNK = EPW // CH
ROWS_PER_TILE = NP // NS

@functools.cache
def _gather_kernel():
    mesh = plsc.VectorSubcoreMesh(core_axis_name="c", subcore_axis_name="s")

    @functools.partial(
        pl.kernel,
        mesh=mesh,
        out_type=jax.ShapeDtypeStruct((EP, D_INT), jnp.float32),
        compiler_params=pltpu.CompilerParams(use_tc_tiling_on_sc=False),
        scratch_types=[
            pltpu.VMEM((CH,), jnp.int32),
            pltpu.VMEM((CH, D_INT), jnp.float32),
            pltpu.SemaphoreType.DMA,
        ],
    )
    def _gather_k(src_hbm, x16_hbm, out_hbm, idx_v, rows_v, sem):
        wid = lax.axis_index("s") * NC + lax.axis_index("c")
        base = wid * EPW

        def body(i, carry):
            off = base + i * CH
            pltpu.sync_copy(src_hbm.at[pl.ds(off, CH)], idx_v)
            pltpu.async_copy(x16_hbm.at[idx_v], rows_v, sem).wait()
            pltpu.sync_copy(rows_v, out_hbm.at[pl.ds(off, CH)])
            return carry

        lax.fori_loop(0, NCHUNK, body, 0)

    return _gather_k


@functools.cache
def _scatter_kernel():
    mesh = plsc.VectorSubcoreMesh(core_axis_name="c", subcore_axis_name="s")

    @functools.partial(
        pl.kernel,
        mesh=mesh,
        out_type=jax.ShapeDtypeStruct((NC, NP, M_OUT), jnp.float32),
        compiler_params=pltpu.CompilerParams(use_tc_tiling_on_sc=False),
        scratch_types=[
            pltpu.VMEM((CH,), jnp.int32),
            pltpu.VMEM((CH, M_OUT), jnp.float32),
            pltpu.VMEM_SHARED((NP, M_OUT), jnp.float32),
            pltpu.SemaphoreType.DMA,
        ],
    )
    def _scatter_k(dst_hbm, msg_hbm, zeros_hbm, out_hbm, idx_v, rows_v,
                   acc_sh, sem):
        cid = lax.axis_index("c")
        sid = lax.axis_index("s")

        @pl.when(sid == 0)
        def _init():
            pltpu.sync_copy(zeros_hbm, acc_sh)

        plsc.subcore_barrier()

        base = cid * (EP // NC) + sid * (EP // NC // NS)

        def body(i, carry):
            off = base + i * CH
            pltpu.sync_copy(dst_hbm.at[pl.ds(off, CH)], idx_v)
            pltpu.sync_copy(msg_hbm.at[pl.ds(off, CH)], rows_v)
            pltpu.sync_copy(rows_v, acc_sh.at[idx_v], add=True)
            return carry

        lax.fori_loop(0, NCHUNK, body, 0)
        plsc.subcore_barrier()

        row0 = sid * ROWS_PER_TILE
        pltpu.sync_copy(acc_sh.at[pl.ds(row0, ROWS_PER_TILE)],
                        out_hbm.at[cid].at[pl.ds(row0, ROWS_PER_TILE)])

    return _scatter_k


def _msg_body(ea_ref, xg_ref, w1_ref, b1_ref, w2_ref, b2_ref, out_ref):
    pid = pl.program_id(0)
    h = jnp.maximum(
        jnp.dot(ea_ref[...].astype(jnp.bfloat16),
                w1_ref[...].astype(jnp.bfloat16),
                preferred_element_type=jnp.float32)
        + b1_ref[...], 0.0)
    w = jnp.dot(h.astype(jnp.bfloat16), w2_ref[...].astype(jnp.bfloat16),
                preferred_element_type=jnp.float32) + b2_ref[...]
    w = w.astype(jnp.bfloat16).astype(jnp.float32)
    xg = xg_ref[...].astype(jnp.bfloat16).astype(jnp.float32)
    acc = xg[:, 0:1] * w[:, 0:M_OUT]
    for i in range(1, D_INT):
        acc = acc + xg[:, i:i + 1] * w[:, i * M_OUT:(i + 1) * M_OUT]
    rid = pid * ET + lax.broadcasted_iota(jnp.int32, (ET, 1), 0)
    out_ref[...] = jnp.where(rid < E, acc, 0.0)


def _elu(v):
    return jnp.where(v > 0, v, jnp.exp(jnp.minimum(v, 0.0)) - 1.0)


def _final_body(agg2_ref, x_ref, n2s_ref, s2g_ref, root_w_ref, conv_b_ref,
                fc1_w_ref, fc1_b_ref, fc2_w_ref, fc2_b_ref,
                fc3_w_ref, fc3_b_ref, out_ref, xc_ref):
    x = x_ref[...]
    x16 = x[:, :D_INT]
    agg = agg2_ref[0] + agg2_ref[1]
    x_h = _elu(agg + jnp.dot(x16.astype(jnp.bfloat16),
                             root_w_ref[...].astype(jnp.bfloat16),
                             preferred_element_type=jnp.float32)
               + conv_b_ref[...])
    ones = jnp.ones((NP, 1), jnp.float32)
    zeros = jnp.zeros((NP, 15), jnp.float32)
    xc_ref[...] = jnp.concatenate([x_h, x[:, D_INT:], ones, zeros], axis=1)

    SB = 512
    CHN = 1024

    def seg_block(sb):
        def body(ci, acc):
            seg_row = n2s_ref[0:1, pl.ds(ci * CHN, CHN)]
            ids = (lax.broadcasted_iota(jnp.int32, (SB, CHN), 0)
                   + (sb * SB)).astype(jnp.float32)
            oh = (ids == seg_row).astype(jnp.float32)
            chunk = xc_ref[pl.ds(ci * CHN, CHN), :]
            return acc + jnp.dot(oh, chunk,
                                 preferred_element_type=jnp.float32,
                                 precision=lax.Precision.HIGHEST)
        return lax.fori_loop(0, NP // CHN, body, jnp.zeros((SB, 64), jnp.float32))

    s1 = jnp.concatenate([seg_block(sb) for sb in range(NSUBP // SB)], axis=0)
    cnt1 = jnp.maximum(s1[:, 48:49], 1.0)
    xs = s1[:, :48] / cnt1

    s2g = s2g_ref[...]
    ids2 = lax.broadcasted_iota(jnp.int32, (NGRAPH, NSUBP), 0).astype(jnp.float32)
    oh2 = (ids2 == s2g[0:1, :]).astype(jnp.float32)
    xs2 = jnp.concatenate([xs, jnp.ones((NSUBP, 1), jnp.float32),
                           jnp.zeros((NSUBP, 15), jnp.float32)], axis=1)
    s2 = jnp.dot(oh2, xs2, preferred_element_type=jnp.float32,
                 precision=lax.Precision.HIGHEST)
    cnt2 = jnp.maximum(s2[:, 48:49], 1.0)
    xg = s2[:, :48] / cnt2

    o = _elu(jnp.dot(xg.astype(jnp.bfloat16),
                     fc1_w_ref[...].astype(jnp.bfloat16),
                     preferred_element_type=jnp.float32) + fc1_b_ref[...])
    o = _elu(jnp.dot(o.astype(jnp.bfloat16),
                     fc2_w_ref[...].astype(jnp.bfloat16),
                     preferred_element_type=jnp.float32) + fc2_b_ref[...])
    o = jnp.dot(o.astype(jnp.bfloat16), fc3_w_ref[...].astype(jnp.bfloat16),
                preferred_element_type=jnp.float32) + fc3_b_ref[...]
    out_ref[...] = o


def kernel(x, edge_index, edge_attr, node_to_subgraph, subgraph_to_graph,
           nn_w1, nn_b1, nn_w2, nn_b2, root_w, conv_b,
           fc1_w, fc1_b, fc2_w, fc2_b, fc3_w, fc3_b):
    src = jnp.pad(edge_index[0], (0, EP - E))
    dst = jnp.pad(edge_index[1], (0, EP - E))
    x16 = x[:, :D_INT]

    xg = _gather_kernel()(src, x16)

    ea_p = jnp.pad(edge_attr, ((0, EP - E), (0, 4)))
    w1_p = jnp.pad(nn_w1, ((0, 4), (0, 0)))
    msg = pl.pallas_call(
        _msg_body,
        grid=(EP // ET,),
        in_specs=[
            pl.BlockSpec((ET, 8), lambda i: (i, 0)),
            pl.BlockSpec((ET, D_INT), lambda i: (i, 0)),
            pl.BlockSpec((8, HID), lambda i: (0, 0)),
            pl.BlockSpec((1, HID), lambda i: (0, 0)),
            pl.BlockSpec((HID, D_INT * M_OUT), lambda i: (0, 0)),
            pl.BlockSpec((1, D_INT * M_OUT), lambda i: (0, 0)),
        ],
        out_specs=pl.BlockSpec((ET, M_OUT), lambda i: (i, 0)),
        out_shape=jax.ShapeDtypeStruct((EP, M_OUT), jnp.float32),
    )(ea_p, xg, w1_p, nn_b1.reshape(1, HID),
      nn_w2, nn_b2.reshape(1, D_INT * M_OUT))

    agg2 = _scatter_kernel()(dst, msg, jnp.zeros((NP, M_OUT), jnp.float32))

    x_p = jnp.pad(x, ((0, NP - N), (0, 0)))
    n2s_f = jnp.tile(
        jnp.pad(node_to_subgraph, (0, NP - N), constant_values=-1)
        .astype(jnp.float32)[None, :], (8, 1))
    s2g_f = jnp.tile(
        jnp.pad(subgraph_to_graph, (0, NSUBP - NSUB), constant_values=-1)
        .astype(jnp.float32)[None, :], (8, 1))
    out = pl.pallas_call(
        _final_body,
        out_shape=jax.ShapeDtypeStruct((NGRAPH, 1), jnp.float32),
        scratch_shapes=[pltpu.VMEM((NP, 64), jnp.float32)],
    )(agg2, x_p, n2s_f, s2g_f, root_w, conv_b.reshape(1, M_OUT),
      fc1_w, fc1_b.reshape(1, 24), fc2_w, fc2_b.reshape(1, 12),
      fc3_w, fc3_b.reshape(1, 1))
    return out.reshape(-1)


# speedup vs baseline: 3.2049x; 1.9765x over previous
import functools

import jax
import jax.numpy as jnp
from jax import lax
from jax.experimental import pallas as pl
from jax.experimental.pallas import tpu as pltpu
from jax.experimental.pallas import tpu_sc as plsc

N = 10000
NPAD = 10240
E = 160000
ET = 2000
NSUB = 2000
NSUBP = 2048
NGRAPH = 64
D_INT = 16
D_EDGE = 4
M_OUT = 32
HID = 128

NC = 2
NS = 16
NW = NC * NS
EPW = E // NW
CH = 1000
NCHUNK = EPW // CH
ROWS_PER_TILE = N // NS

@functools.cache
def _gather_kernel():
    mesh = plsc.VectorSubcoreMesh(core_axis_name="c", subcore_axis_name="s")

    @functools.partial(
        pl.kernel,
        mesh=mesh,
        out_type=jax.ShapeDtypeStruct((E, D_INT), jnp.float32),
        compiler_params=pltpu.CompilerParams(use_tc_tiling_on_sc=False),
        scratch_types=[
            pltpu.VMEM((CH,), jnp.int32),
            pltpu.VMEM((CH, D_INT), jnp.float32),
            pltpu.SemaphoreType.DMA,
        ],
    )
    def _gather_k(src_hbm, x16_hbm, out_hbm, idx_v, rows_v, sem):
        wid = lax.axis_index("s") * NC + lax.axis_index("c")
        base = wid * EPW

        def body(i, carry):
            off = base + i * CH
            pltpu.sync_copy(src_hbm.at[pl.ds(off, CH)], idx_v)
            pltpu.async_copy(x16_hbm.at[idx_v], rows_v, sem).wait()
            pltpu.sync_copy(rows_v, out_hbm.at[pl.ds(off, CH)])
            return carry

        lax.fori_loop(0, NCHUNK, body, 0)

    return _gather_k


@functools.cache
def _scatter_kernel():
    mesh = plsc.VectorSubcoreMesh(core_axis_name="c", subcore_axis_name="s")

    @functools.partial(
        pl.kernel,
        mesh=mesh,
        out_type=jax.ShapeDtypeStruct((NC, N, M_OUT), jnp.float32),
        compiler_params=pltpu.CompilerParams(use_tc_tiling_on_sc=False),
        scratch_types=[
            pltpu.VMEM((CH,), jnp.int32),
            pltpu.VMEM((CH, M_OUT), jnp.float32),
            pltpu.VMEM_SHARED((N, M_OUT), jnp.float32),
            pltpu.SemaphoreType.DMA,
        ],
    )
    def _scatter_k(dst_hbm, msg_hbm, zeros_hbm, out_hbm, idx_v, rows_v,
                   acc_sh, sem):
        cid = lax.axis_index("c")
        sid = lax.axis_index("s")

        @pl.when(sid == 0)
        def _init():
            pltpu.sync_copy(zeros_hbm, acc_sh)

        plsc.subcore_barrier()

        base = cid * (E // NC) + sid * (E // NC // NS)

        def body(i, carry):
            off = base + i * CH
            pltpu.sync_copy(dst_hbm.at[pl.ds(off, CH)], idx_v)
            pltpu.sync_copy(msg_hbm.at[pl.ds(off, CH)], rows_v)
            pltpu.sync_copy(rows_v, acc_sh.at[idx_v], add=True)
            return carry

        lax.fori_loop(0, NCHUNK, body, 0)
        plsc.subcore_barrier()

        row0 = sid * ROWS_PER_TILE
        pltpu.sync_copy(acc_sh.at[pl.ds(row0, ROWS_PER_TILE)],
                        out_hbm.at[cid].at[pl.ds(row0, ROWS_PER_TILE)])

    return _scatter_k


def _msg_body(ea_ref, xg_ref, w1_ref, b1_ref, w2_ref, b2_ref, r_ref, out_ref):
    h = jnp.maximum(
        jnp.dot(ea_ref[...].astype(jnp.bfloat16),
                w1_ref[...].astype(jnp.bfloat16),
                preferred_element_type=jnp.float32)
        + b1_ref[...], 0.0)
    w = jnp.dot(h.astype(jnp.bfloat16), w2_ref[...].astype(jnp.bfloat16),
                preferred_element_type=jnp.float32) + b2_ref[...]
    w = w.astype(jnp.bfloat16).astype(jnp.float32)
    xg_e = jnp.dot(xg_ref[...].astype(jnp.bfloat16), r_ref[...],
                   preferred_element_type=jnp.float32)
    s = xg_e * w
    s = s[:, :256] + s[:, 256:]
    s = s[:, :128] + s[:, 128:]
    s = s[:, :64] + s[:, 64:]
    out_ref[...] = s[:, :32] + s[:, 32:]


def _elu(v):
    return jnp.where(v > 0, v, jnp.exp(jnp.minimum(v, 0.0)) - 1.0)


def _final_body(agg2_ref, x_ref, n2s_ref, s2g_ref, root_w_ref, conv_b_ref,
                fc1_w_ref, fc1_b_ref, fc2_w_ref, fc2_b_ref,
                fc3_w_ref, fc3_b_ref, out_ref, xc_ref):
    x = x_ref[...]
    x16 = x[:, :D_INT]
    agg = agg2_ref[0] + agg2_ref[1]
    x_h = _elu(agg + jnp.dot(x16.astype(jnp.bfloat16),
                             root_w_ref[...].astype(jnp.bfloat16),
                             preferred_element_type=jnp.float32)
               + conv_b_ref[...])
    ones = jnp.ones((N, 1), jnp.float32)
    zeros = jnp.zeros((N, 15), jnp.float32)
    xc_ref[:N, :] = jnp.concatenate([x_h, x[:, D_INT:], ones, zeros], axis=1)
    xc_ref[N:, :] = jnp.zeros((NPAD - N, 64), jnp.float32)

    SB = 512
    CHN = 1024

    def seg_block(sb):
        def body(ci, acc):
            seg_row = n2s_ref[0:1, pl.ds(ci * CHN, CHN)]
            ids = (lax.broadcasted_iota(jnp.int32, (SB, CHN), 0)
                   + (sb * SB)).astype(jnp.float32)
            oh = (ids == seg_row).astype(jnp.float32)
            chunk = xc_ref[pl.ds(ci * CHN, CHN), :]
            return acc + jnp.dot(oh, chunk,
                                 preferred_element_type=jnp.float32,
                                 precision=lax.Precision.HIGHEST)
        return lax.fori_loop(0, NPAD // CHN, body,
                             jnp.zeros((SB, 64), jnp.float32))

    s1 = jnp.concatenate([seg_block(sb) for sb in range(NSUBP // SB)], axis=0)
    cnt1 = jnp.maximum(s1[:, 48:49], 1.0)
    xs = s1[:, :48] / cnt1

    s2g = s2g_ref[...]
    ids2 = lax.broadcasted_iota(jnp.int32, (NGRAPH, NSUBP), 0).astype(jnp.float32)
    oh2 = (ids2 == s2g[0:1, :]).astype(jnp.float32)
    xs2 = jnp.concatenate([xs, jnp.ones((NSUBP, 1), jnp.float32),
                           jnp.zeros((NSUBP, 15), jnp.float32)], axis=1)
    s2 = jnp.dot(oh2, xs2, preferred_element_type=jnp.float32,
                 precision=lax.Precision.HIGHEST)
    cnt2 = jnp.maximum(s2[:, 48:49], 1.0)
    xg = s2[:, :48] / cnt2

    o = _elu(jnp.dot(xg.astype(jnp.bfloat16),
                     fc1_w_ref[...].astype(jnp.bfloat16),
                     preferred_element_type=jnp.float32) + fc1_b_ref[...])
    o = _elu(jnp.dot(o.astype(jnp.bfloat16),
                     fc2_w_ref[...].astype(jnp.bfloat16),
                     preferred_element_type=jnp.float32) + fc2_b_ref[...])
    o = jnp.dot(o.astype(jnp.bfloat16), fc3_w_ref[...].astype(jnp.bfloat16),
                preferred_element_type=jnp.float32) + fc3_b_ref[...]
    out_ref[...] = o


def kernel(x, edge_index, edge_attr, node_to_subgraph, subgraph_to_graph,
           nn_w1, nn_b1, nn_w2, nn_b2, root_w, conv_b,
           fc1_w, fc1_b, fc2_w, fc2_b, fc3_w, fc3_b):
    src = edge_index[0]
    dst = edge_index[1]
    x16 = x[:, :D_INT]

    xg = _gather_kernel()(src, x16)

    r_oh = (jnp.arange(D_INT * M_OUT)[None, :] // M_OUT
            == jnp.arange(D_INT)[:, None]).astype(jnp.bfloat16)
    msg = pl.pallas_call(
        _msg_body,
        grid=(E // ET,),
        in_specs=[
            pl.BlockSpec((ET, D_EDGE), lambda i: (i, 0)),
            pl.BlockSpec((ET, D_INT), lambda i: (i, 0)),
            pl.BlockSpec((D_EDGE, HID), lambda i: (0, 0)),
            pl.BlockSpec((1, HID), lambda i: (0, 0)),
            pl.BlockSpec((HID, D_INT * M_OUT), lambda i: (0, 0)),
            pl.BlockSpec((1, D_INT * M_OUT), lambda i: (0, 0)),
            pl.BlockSpec((D_INT, D_INT * M_OUT), lambda i: (0, 0)),
        ],
        out_specs=pl.BlockSpec((ET, M_OUT), lambda i: (i, 0)),
        out_shape=jax.ShapeDtypeStruct((E, M_OUT), jnp.float32),
    )(edge_attr, xg, nn_w1, nn_b1.reshape(1, HID),
      nn_w2, nn_b2.reshape(1, D_INT * M_OUT), r_oh)

    agg2 = _scatter_kernel()(dst, msg, jnp.zeros((N, M_OUT), jnp.float32))

    n2s_f = jnp.pad(node_to_subgraph, (0, NPAD - N),
                    constant_values=-1).astype(jnp.float32).reshape(1, NPAD)
    s2g_f = jnp.pad(subgraph_to_graph, (0, NSUBP - NSUB),
                    constant_values=-1).astype(jnp.float32).reshape(1, NSUBP)
    out = pl.pallas_call(
        _final_body,
        out_shape=jax.ShapeDtypeStruct((NGRAPH, 1), jnp.float32),
        scratch_shapes=[pltpu.VMEM((NPAD, 64), jnp.float32)],
    )(agg2, x, n2s_f, s2g_f, root_w, conv_b.reshape(1, M_OUT),
      fc1_w, fc1_b.reshape(1, 24), fc2_w, fc2_b.reshape(1, 12),
      fc3_w, fc3_b.reshape(1, 1))
    return out.reshape(-1)


# speedup vs baseline: 3.6694x; 1.1449x over previous
import functools

import jax
import jax.numpy as jnp
from jax import lax
from jax.experimental import pallas as pl
from jax.experimental.pallas import tpu as pltpu
from jax.experimental.pallas import tpu_sc as plsc

N = 10000
NPAD = 10240
E = 160000
EP = 163840
ET = 2048
NSUB = 2000
NSUBP = 2048
NGRAPH = 64
D_INT = 16
D_EDGE = 4
M_OUT = 32
HID = 128

NC = 2
NS = 16
NW = NC * NS
EPW = EP // NW
CH = 1024
NCHUNK = EPW // CH
ROWS_PER_TILE = N // NS

@functools.cache
def _gather_kernel():
    mesh = plsc.VectorSubcoreMesh(core_axis_name="c", subcore_axis_name="s")

    @functools.partial(
        pl.kernel,
        mesh=mesh,
        out_type=jax.ShapeDtypeStruct((EP, D_INT), jnp.float32),
        compiler_params=pltpu.CompilerParams(use_tc_tiling_on_sc=False),
        scratch_types=[
            pltpu.VMEM((CH,), jnp.int32),
            pltpu.VMEM((CH, D_INT), jnp.float32),
            pltpu.SemaphoreType.DMA,
        ],
    )
    def _gather_k(src_hbm, x16_hbm, out_hbm, idx_v, rows_v, sem):
        wid = lax.axis_index("s") * NC + lax.axis_index("c")
        base = wid * EPW

        def body(i, carry):
            off = base + i * CH
            pltpu.sync_copy(src_hbm.at[pl.ds(off, CH)], idx_v)
            pltpu.async_copy(x16_hbm.at[idx_v], rows_v, sem).wait()
            pltpu.sync_copy(rows_v, out_hbm.at[pl.ds(off, CH)])
            return carry

        lax.fori_loop(0, NCHUNK, body, 0)

    return _gather_k


@functools.cache
def _scatter_kernel():
    mesh = plsc.VectorSubcoreMesh(core_axis_name="c", subcore_axis_name="s")

    @functools.partial(
        pl.kernel,
        mesh=mesh,
        out_type=jax.ShapeDtypeStruct((NC, N, M_OUT), jnp.float32),
        compiler_params=pltpu.CompilerParams(use_tc_tiling_on_sc=False),
        scratch_types=[
            pltpu.VMEM((CH,), jnp.int32),
            pltpu.VMEM((CH, M_OUT), jnp.float32),
            pltpu.VMEM_SHARED((N, M_OUT), jnp.float32),
            pltpu.SemaphoreType.DMA,
        ],
    )
    def _scatter_k(dst_hbm, msg_hbm, zeros_hbm, out_hbm, idx_v, rows_v,
                   acc_sh, sem):
        cid = lax.axis_index("c")
        sid = lax.axis_index("s")

        @pl.when(sid == 0)
        def _init():
            pltpu.sync_copy(zeros_hbm, acc_sh)

        plsc.subcore_barrier()

        base = cid * (EP // NC) + sid * (EP // NC // NS)

        def body(i, carry):
            off = base + i * CH
            pltpu.sync_copy(dst_hbm.at[pl.ds(off, CH)], idx_v)
            pltpu.sync_copy(msg_hbm.at[pl.ds(off, CH)], rows_v)
            pltpu.sync_copy(rows_v, acc_sh.at[idx_v], add=True)
            return carry

        lax.fori_loop(0, NCHUNK, body, 0)
        plsc.subcore_barrier()

        row0 = sid * ROWS_PER_TILE
        pltpu.sync_copy(acc_sh.at[pl.ds(row0, ROWS_PER_TILE)],
                        out_hbm.at[cid].at[pl.ds(row0, ROWS_PER_TILE)])

    return _scatter_k


def _msg_body(ea_ref, xg_ref, w1_ref, b1_ref, w2_ref, b2_ref, r_ref, out_ref):
    pid = pl.program_id(0)
    h = jnp.maximum(
        lax.dot_general(ea_ref[...].astype(jnp.bfloat16),
                        w1_ref[...].astype(jnp.bfloat16),
                        (((0,), (0,)), ((), ())),
                        preferred_element_type=jnp.float32)
        + b1_ref[...], 0.0)
    w = jnp.dot(h.astype(jnp.bfloat16), w2_ref[...].astype(jnp.bfloat16),
                preferred_element_type=jnp.float32) + b2_ref[...]
    w = w.astype(jnp.bfloat16).astype(jnp.float32)
    xg_e = jnp.dot(xg_ref[...].astype(jnp.bfloat16), r_ref[...],
                   preferred_element_type=jnp.float32)
    s = xg_e * w
    s = s[:, :256] + s[:, 256:]
    s = s[:, :128] + s[:, 128:]
    s = s[:, :64] + s[:, 64:]
    s = s[:, :32] + s[:, 32:]
    rid = pid * ET + lax.broadcasted_iota(jnp.int32, (ET, 1), 0)
    out_ref[...] = jnp.where(rid < E, s, 0.0)


def _elu(v):
    return jnp.where(v > 0, v, jnp.exp(jnp.minimum(v, 0.0)) - 1.0)


def _final_body(agg2_ref, x_ref, n2s_ref, s2g_ref, root_w_ref, conv_b_ref,
                fc1_w_ref, fc1_b_ref, fc2_w_ref, fc2_b_ref,
                fc3_w_ref, fc3_b_ref, out_ref, xch_ref, xcl_ref):
    x = x_ref[...]
    x16 = x[:, :D_INT]
    agg = agg2_ref[0] + agg2_ref[1]
    x_h = _elu(agg + jnp.dot(x16.astype(jnp.bfloat16),
                             root_w_ref[...].astype(jnp.bfloat16),
                             preferred_element_type=jnp.float32)
               + conv_b_ref[...])
    ones = jnp.ones((N, 1), jnp.float32)
    zeros = jnp.zeros((N, 15), jnp.float32)
    xc = jnp.concatenate([x_h, x[:, D_INT:], ones, zeros], axis=1)
    xc_hi = xc.astype(jnp.bfloat16)
    xc_lo = (xc - xc_hi.astype(jnp.float32)).astype(jnp.bfloat16)
    zpad = jnp.zeros((NPAD - N, 64), jnp.bfloat16)
    xch_ref[...] = jnp.concatenate([xc_hi, zpad], axis=0)
    xcl_ref[...] = jnp.concatenate([xc_lo, zpad], axis=0)

    SB = 512
    CHN = 1024

    def seg_block(sb):
        def body(ci, acc):
            seg_row = n2s_ref[0:1, pl.ds(ci * CHN, CHN)]
            ids = (lax.broadcasted_iota(jnp.int32, (SB, CHN), 0)
                   + (sb * SB)).astype(jnp.float32)
            oh = (ids == seg_row).astype(jnp.bfloat16)
            hi = xch_ref[pl.ds(ci * CHN, CHN), :]
            lo = xcl_ref[pl.ds(ci * CHN, CHN), :]
            return (acc
                    + jnp.dot(oh, hi, preferred_element_type=jnp.float32)
                    + jnp.dot(oh, lo, preferred_element_type=jnp.float32))
        return lax.fori_loop(0, NPAD // CHN, body,
                             jnp.zeros((SB, 64), jnp.float32))

    s1 = jnp.concatenate([seg_block(sb) for sb in range(NSUBP // SB)], axis=0)
    cnt1 = jnp.maximum(s1[:, 48:49], 1.0)
    xs = s1[:, :48] / cnt1

    s2g = s2g_ref[...]
    ids2 = lax.broadcasted_iota(jnp.int32, (NGRAPH, NSUBP), 0).astype(jnp.float32)
    oh2 = (ids2 == s2g[0:1, :]).astype(jnp.bfloat16)
    xs2 = jnp.concatenate([xs, jnp.ones((NSUBP, 1), jnp.float32),
                           jnp.zeros((NSUBP, 15), jnp.float32)], axis=1)
    xs2_hi = xs2.astype(jnp.bfloat16)
    xs2_lo = (xs2 - xs2_hi.astype(jnp.float32)).astype(jnp.bfloat16)
    s2 = (jnp.dot(oh2, xs2_hi, preferred_element_type=jnp.float32)
          + jnp.dot(oh2, xs2_lo, preferred_element_type=jnp.float32))
    cnt2 = jnp.maximum(s2[:, 48:49], 1.0)
    xg = s2[:, :48] / cnt2

    o = _elu(jnp.dot(xg.astype(jnp.bfloat16),
                     fc1_w_ref[...].astype(jnp.bfloat16),
                     preferred_element_type=jnp.float32) + fc1_b_ref[...])
    o = _elu(jnp.dot(o.astype(jnp.bfloat16),
                     fc2_w_ref[...].astype(jnp.bfloat16),
                     preferred_element_type=jnp.float32) + fc2_b_ref[...])
    o = jnp.dot(o.astype(jnp.bfloat16), fc3_w_ref[...].astype(jnp.bfloat16),
                preferred_element_type=jnp.float32) + fc3_b_ref[...]
    out_ref[...] = o


def kernel(x, edge_index, edge_attr, node_to_subgraph, subgraph_to_graph,
           nn_w1, nn_b1, nn_w2, nn_b2, root_w, conv_b,
           fc1_w, fc1_b, fc2_w, fc2_b, fc3_w, fc3_b):
    src = jnp.pad(edge_index[0], (0, EP - E))
    dst = jnp.pad(edge_index[1], (0, EP - E))
    x16 = x[:, :D_INT]

    xg = _gather_kernel()(src, x16)

    ea_t = jnp.pad(edge_attr.T, ((0, 0), (0, EP - E)))
    r_oh = (jnp.arange(D_INT * M_OUT)[None, :] // M_OUT
            == jnp.arange(D_INT)[:, None]).astype(jnp.bfloat16)
    msg = pl.pallas_call(
        _msg_body,
        grid=(EP // ET,),
        in_specs=[
            pl.BlockSpec((D_EDGE, ET), lambda i: (0, i)),
            pl.BlockSpec((ET, D_INT), lambda i: (i, 0)),
            pl.BlockSpec((D_EDGE, HID), lambda i: (0, 0)),
            pl.BlockSpec((1, HID), lambda i: (0, 0)),
            pl.BlockSpec((HID, D_INT * M_OUT), lambda i: (0, 0)),
            pl.BlockSpec((1, D_INT * M_OUT), lambda i: (0, 0)),
            pl.BlockSpec((D_INT, D_INT * M_OUT), lambda i: (0, 0)),
        ],
        out_specs=pl.BlockSpec((ET, M_OUT), lambda i: (i, 0)),
        out_shape=jax.ShapeDtypeStruct((EP, M_OUT), jnp.float32),
    )(ea_t, xg, nn_w1, nn_b1.reshape(1, HID),
      nn_w2, nn_b2.reshape(1, D_INT * M_OUT), r_oh)

    agg2 = _scatter_kernel()(dst, msg, jnp.zeros((N, M_OUT), jnp.float32))

    n2s_f = jnp.pad(node_to_subgraph, (0, NPAD - N),
                    constant_values=-1).astype(jnp.float32).reshape(1, NPAD)
    s2g_f = jnp.pad(subgraph_to_graph, (0, NSUBP - NSUB),
                    constant_values=-1).astype(jnp.float32).reshape(1, NSUBP)
    out = pl.pallas_call(
        _final_body,
        out_shape=jax.ShapeDtypeStruct((NGRAPH, 1), jnp.float32),
        scratch_shapes=[pltpu.VMEM((NPAD, 64), jnp.bfloat16),
                        pltpu.VMEM((NPAD, 64), jnp.bfloat16)],
    )(agg2, x, n2s_f, s2g_f, root_w, conv_b.reshape(1, M_OUT),
      fc1_w, fc1_b.reshape(1, 24), fc2_w, fc2_b.reshape(1, 12),
      fc3_w, fc3_b.reshape(1, 1))
    return out.reshape(-1)


# speedup vs baseline: 3.7240x; 1.0149x over previous
import functools

import jax
import jax.numpy as jnp
from jax import lax
from jax.experimental import pallas as pl
from jax.experimental.pallas import tpu as pltpu
from jax.experimental.pallas import tpu_sc as plsc

N = 10000
NPAD = 10240
E = 160000
EP = 163840
ET = 2048
NSUB = 2000
NSUBP = 2048
NGRAPH = 64
D_INT = 16
D_EDGE = 4
M_OUT = 32
HID = 128

NC = 2
NS = 16
NW = NC * NS
EPW = EP // NW
CH = 1024
NCHUNK = EPW // CH
ROWS_PER_TILE = N // NS
NACC = N + 8

@functools.cache
def _gather_kernel():
    mesh = plsc.VectorSubcoreMesh(core_axis_name="c", subcore_axis_name="s")

    @functools.partial(
        pl.kernel,
        mesh=mesh,
        out_type=jax.ShapeDtypeStruct((EP, D_INT), jnp.bfloat16),
        compiler_params=pltpu.CompilerParams(use_tc_tiling_on_sc=False),
        scratch_types=[
            pltpu.VMEM((CH,), jnp.int32),
            pltpu.VMEM((CH, D_INT), jnp.bfloat16),
            pltpu.SemaphoreType.DMA,
        ],
    )
    def _gather_k(src_hbm, x16_hbm, out_hbm, idx_v, rows_v, sem):
        wid = lax.axis_index("s") * NC + lax.axis_index("c")
        base = wid * EPW

        def body(i, carry):
            off = base + i * CH
            pltpu.sync_copy(src_hbm.at[pl.ds(off, CH)], idx_v)
            pltpu.async_copy(x16_hbm.at[idx_v], rows_v, sem).wait()
            pltpu.sync_copy(rows_v, out_hbm.at[pl.ds(off, CH)])
            return carry

        lax.fori_loop(0, NCHUNK, body, 0)

    return _gather_k


@functools.cache
def _scatter_kernel():
    mesh = plsc.VectorSubcoreMesh(core_axis_name="c", subcore_axis_name="s")

    @functools.partial(
        pl.kernel,
        mesh=mesh,
        out_type=jax.ShapeDtypeStruct((NC, N, M_OUT), jnp.float32),
        compiler_params=pltpu.CompilerParams(use_tc_tiling_on_sc=False),
        scratch_types=[
            pltpu.VMEM((CH,), jnp.int32),
            pltpu.VMEM((CH, M_OUT), jnp.float32),
            pltpu.VMEM_SHARED((NACC, M_OUT), jnp.float32),
            pltpu.SemaphoreType.DMA,
        ],
    )
    def _scatter_k(dst_hbm, msg_hbm, zeros_hbm, out_hbm, idx_v, rows_v,
                   acc_sh, sem):
        cid = lax.axis_index("c")
        sid = lax.axis_index("s")

        @pl.when(sid == 0)
        def _init():
            pltpu.sync_copy(zeros_hbm, acc_sh)

        plsc.subcore_barrier()

        base = cid * (EP // NC) + sid * (EP // NC // NS)

        def body(i, carry):
            off = base + i * CH
            pltpu.sync_copy(dst_hbm.at[pl.ds(off, CH)], idx_v)
            pltpu.sync_copy(msg_hbm.at[pl.ds(off, CH)], rows_v)
            pltpu.sync_copy(rows_v, acc_sh.at[idx_v], add=True)
            return carry

        lax.fori_loop(0, NCHUNK, body, 0)
        plsc.subcore_barrier()

        row0 = sid * ROWS_PER_TILE
        pltpu.sync_copy(acc_sh.at[pl.ds(row0, ROWS_PER_TILE)],
                        out_hbm.at[cid].at[pl.ds(row0, ROWS_PER_TILE)])

    return _scatter_k


def _msg_body(ea_ref, xg_ref, w1_ref, b1_ref, w2_ref, b2_ref, r_ref, out_ref):
    h = jnp.maximum(
        lax.dot_general(ea_ref[...].astype(jnp.bfloat16),
                        w1_ref[...].astype(jnp.bfloat16),
                        (((0,), (0,)), ((), ())),
                        preferred_element_type=jnp.float32)
        + b1_ref[...], 0.0)
    w = jnp.dot(h.astype(jnp.bfloat16), w2_ref[...].astype(jnp.bfloat16),
                preferred_element_type=jnp.float32) + b2_ref[...]
    w = w.astype(jnp.bfloat16).astype(jnp.float32)
    xg_e = jnp.dot(xg_ref[...], r_ref[...],
                   preferred_element_type=jnp.float32)
    s = xg_e * w
    s = s[:, :256] + s[:, 256:]
    s = s[:, :128] + s[:, 128:]
    s = s[:, :64] + s[:, 64:]
    out_ref[...] = s[:, :32] + s[:, 32:]


def _elu(v):
    return jnp.where(v > 0, v, jnp.exp(jnp.minimum(v, 0.0)) - 1.0)


def _final_body(agg2_ref, x_ref, n2s_ref, s2g_ref, root_w_ref, conv_b_ref,
                fc1_w_ref, fc1_b_ref, fc2_w_ref, fc2_b_ref,
                fc3_w_ref, fc3_b_ref, out_ref, xch_ref, xcl_ref):
    x = x_ref[...]
    x16 = x[:, :D_INT]
    agg = agg2_ref[0] + agg2_ref[1]
    x_h = _elu(agg + jnp.dot(x16.astype(jnp.bfloat16),
                             root_w_ref[...].astype(jnp.bfloat16),
                             preferred_element_type=jnp.float32)
               + conv_b_ref[...])
    ones = jnp.ones((N, 1), jnp.float32)
    zeros = jnp.zeros((N, 15), jnp.float32)
    xc = jnp.concatenate([x_h, x[:, D_INT:], ones, zeros], axis=1)
    xc_hi = xc.astype(jnp.bfloat16)
    xc_lo = (xc - xc_hi.astype(jnp.float32)).astype(jnp.bfloat16)
    zpad = jnp.zeros((NPAD - N, 64), jnp.bfloat16)
    xch_ref[...] = jnp.concatenate([xc_hi, zpad], axis=0)
    xcl_ref[...] = jnp.concatenate([xc_lo, zpad], axis=0)

    SB = 512
    CHN = 1024

    def seg_block(sb):
        def body(ci, acc):
            seg_row = n2s_ref[0:1, pl.ds(ci * CHN, CHN)]
            ids = (lax.broadcasted_iota(jnp.int32, (SB, CHN), 0)
                   + (sb * SB)).astype(jnp.float32)
            oh = (ids == seg_row).astype(jnp.bfloat16)
            hi = xch_ref[pl.ds(ci * CHN, CHN), :]
            lo = xcl_ref[pl.ds(ci * CHN, CHN), :]
            return (acc
                    + jnp.dot(oh, hi, preferred_element_type=jnp.float32)
                    + jnp.dot(oh, lo, preferred_element_type=jnp.float32))
        return lax.fori_loop(0, NPAD // CHN, body,
                             jnp.zeros((SB, 64), jnp.float32))

    s1 = jnp.concatenate([seg_block(sb) for sb in range(NSUBP // SB)], axis=0)
    cnt1 = jnp.maximum(s1[:, 48:49], 1.0)
    xs = s1[:, :48] / cnt1

    s2g = s2g_ref[...]
    ids2 = lax.broadcasted_iota(jnp.int32, (NGRAPH, NSUBP), 0).astype(jnp.float32)
    oh2 = (ids2 == s2g[0:1, :]).astype(jnp.bfloat16)
    xs2 = jnp.concatenate([xs, jnp.ones((NSUBP, 1), jnp.float32),
                           jnp.zeros((NSUBP, 15), jnp.float32)], axis=1)
    xs2_hi = xs2.astype(jnp.bfloat16)
    xs2_lo = (xs2 - xs2_hi.astype(jnp.float32)).astype(jnp.bfloat16)
    s2 = (jnp.dot(oh2, xs2_hi, preferred_element_type=jnp.float32)
          + jnp.dot(oh2, xs2_lo, preferred_element_type=jnp.float32))
    cnt2 = jnp.maximum(s2[:, 48:49], 1.0)
    xg = s2[:, :48] / cnt2

    o = _elu(jnp.dot(xg.astype(jnp.bfloat16),
                     fc1_w_ref[...].astype(jnp.bfloat16),
                     preferred_element_type=jnp.float32) + fc1_b_ref[...])
    o = _elu(jnp.dot(o.astype(jnp.bfloat16),
                     fc2_w_ref[...].astype(jnp.bfloat16),
                     preferred_element_type=jnp.float32) + fc2_b_ref[...])
    o = jnp.dot(o.astype(jnp.bfloat16), fc3_w_ref[...].astype(jnp.bfloat16),
                preferred_element_type=jnp.float32) + fc3_b_ref[...]
    out_ref[...] = o


def kernel(x, edge_index, edge_attr, node_to_subgraph, subgraph_to_graph,
           nn_w1, nn_b1, nn_w2, nn_b2, root_w, conv_b,
           fc1_w, fc1_b, fc2_w, fc2_b, fc3_w, fc3_b):
    src = jnp.pad(edge_index[0], (0, EP - E))
    dst = jnp.pad(edge_index[1], (0, EP - E), constant_values=N)
    x16 = x[:, :D_INT]

    xg = _gather_kernel()(src, x16.astype(jnp.bfloat16))

    ea_t = jnp.pad(edge_attr.T, ((0, 0), (0, EP - E)))
    r_oh = (jnp.arange(D_INT * M_OUT)[None, :] // M_OUT
            == jnp.arange(D_INT)[:, None]).astype(jnp.bfloat16)
    msg = pl.pallas_call(
        _msg_body,
        grid=(EP // ET,),
        in_specs=[
            pl.BlockSpec((D_EDGE, ET), lambda i: (0, i)),
            pl.BlockSpec((ET, D_INT), lambda i: (i, 0)),
            pl.BlockSpec((D_EDGE, HID), lambda i: (0, 0)),
            pl.BlockSpec((1, HID), lambda i: (0, 0)),
            pl.BlockSpec((HID, D_INT * M_OUT), lambda i: (0, 0)),
            pl.BlockSpec((1, D_INT * M_OUT), lambda i: (0, 0)),
            pl.BlockSpec((D_INT, D_INT * M_OUT), lambda i: (0, 0)),
        ],
        out_specs=pl.BlockSpec((ET, M_OUT), lambda i: (i, 0)),
        out_shape=jax.ShapeDtypeStruct((EP, M_OUT), jnp.float32),
    )(ea_t, xg, nn_w1, nn_b1.reshape(1, HID),
      nn_w2, nn_b2.reshape(1, D_INT * M_OUT), r_oh)

    agg2 = _scatter_kernel()(dst, msg, jnp.zeros((NACC, M_OUT), jnp.float32))

    n2s_f = jnp.pad(node_to_subgraph, (0, NPAD - N),
                    constant_values=-1).astype(jnp.float32).reshape(1, NPAD)
    s2g_f = jnp.pad(subgraph_to_graph, (0, NSUBP - NSUB),
                    constant_values=-1).astype(jnp.float32).reshape(1, NSUBP)
    out = pl.pallas_call(
        _final_body,
        out_shape=jax.ShapeDtypeStruct((NGRAPH, 1), jnp.float32),
        scratch_shapes=[pltpu.VMEM((NPAD, 64), jnp.bfloat16),
                        pltpu.VMEM((NPAD, 64), jnp.bfloat16)],
    )(agg2, x, n2s_f, s2g_f, root_w, conv_b.reshape(1, M_OUT),
      fc1_w, fc1_b.reshape(1, 24), fc2_w, fc2_b.reshape(1, 12),
      fc3_w, fc3_b.reshape(1, 1))
    return out.reshape(-1)


# speedup vs baseline: 3.7896x; 1.0176x over previous
import functools

import jax
import jax.numpy as jnp
from jax import lax
from jax.experimental import pallas as pl
from jax.experimental.pallas import tpu as pltpu
from jax.experimental.pallas import tpu_sc as plsc

N = 10000
NPAD = 10240
E = 160000
EP = 163840
ET = 2048
NSUB = 2000
NSUBP = 2048
NGRAPH = 64
D_INT = 16
D_EDGE = 4
M_OUT = 32
HID = 128

NC = 2
NS = 16
NW = NC * NS
EPW = EP // NW
CHG = EPW
CHS = EPW // 2
ROWS_PER_TILE = N // NS
NACC = N + 8

@functools.cache
def _gather_kernel():
    mesh = plsc.VectorSubcoreMesh(core_axis_name="c", subcore_axis_name="s")

    @functools.partial(
        pl.kernel,
        mesh=mesh,
        out_type=jax.ShapeDtypeStruct((EP, D_INT), jnp.bfloat16),
        compiler_params=pltpu.CompilerParams(use_tc_tiling_on_sc=False),
        scratch_types=[
            pltpu.VMEM((CHG,), jnp.int32),
            pltpu.VMEM((CHG, D_INT), jnp.bfloat16),
            pltpu.SemaphoreType.DMA,
        ],
    )
    def _gather_k(src_hbm, x16_hbm, out_hbm, idx_v, rows_v, sem):
        wid = lax.axis_index("s") * NC + lax.axis_index("c")
        base = wid * EPW
        pltpu.sync_copy(src_hbm.at[pl.ds(base, CHG)], idx_v)
        pltpu.async_copy(x16_hbm.at[idx_v], rows_v, sem).wait()
        pltpu.sync_copy(rows_v, out_hbm.at[pl.ds(base, CHG)])

    return _gather_k


@functools.cache
def _scatter_kernel():
    mesh = plsc.VectorSubcoreMesh(core_axis_name="c", subcore_axis_name="s")

    @functools.partial(
        pl.kernel,
        mesh=mesh,
        out_type=jax.ShapeDtypeStruct((NC, N, M_OUT), jnp.float32),
        compiler_params=pltpu.CompilerParams(use_tc_tiling_on_sc=False),
        scratch_types=[
            pltpu.VMEM((CHS,), jnp.int32),
            pltpu.VMEM((CHS, M_OUT), jnp.float32),
            pltpu.VMEM_SHARED((NACC, M_OUT), jnp.float32),
            pltpu.SemaphoreType.DMA,
        ],
    )
    def _scatter_k(dst_hbm, msg_hbm, zeros_hbm, out_hbm, idx_v, rows_v,
                   acc_sh, sem):
        cid = lax.axis_index("c")
        sid = lax.axis_index("s")

        @pl.when(sid == 0)
        def _init():
            pltpu.sync_copy(zeros_hbm, acc_sh)

        plsc.subcore_barrier()

        base = cid * (EP // NC) + sid * (EP // NC // NS)

        def body(i, carry):
            off = base + i * CHS
            pltpu.sync_copy(dst_hbm.at[pl.ds(off, CHS)], idx_v)
            pltpu.sync_copy(msg_hbm.at[pl.ds(off, CHS)], rows_v)
            pltpu.sync_copy(rows_v, acc_sh.at[idx_v], add=True)
            return carry

        lax.fori_loop(0, 2, body, 0)
        plsc.subcore_barrier()

        row0 = sid * ROWS_PER_TILE
        pltpu.sync_copy(acc_sh.at[pl.ds(row0, ROWS_PER_TILE)],
                        out_hbm.at[cid].at[pl.ds(row0, ROWS_PER_TILE)])

    return _scatter_k


def _msg_body(ea_ref, xg_ref, w1_ref, b1_ref, w2_ref, b2_ref, r_ref, out_ref):
    h = jnp.maximum(
        lax.dot_general(ea_ref[...].astype(jnp.bfloat16),
                        w1_ref[...].astype(jnp.bfloat16),
                        (((0,), (0,)), ((), ())),
                        preferred_element_type=jnp.float32)
        + b1_ref[...], 0.0)
    w = jnp.dot(h.astype(jnp.bfloat16), w2_ref[...].astype(jnp.bfloat16),
                preferred_element_type=jnp.float32) + b2_ref[...]
    w = w.astype(jnp.bfloat16).astype(jnp.float32)
    xg_e = jnp.dot(xg_ref[...], r_ref[...],
                   preferred_element_type=jnp.float32)
    s = xg_e * w
    s = s[:, :256] + s[:, 256:]
    s = s[:, :128] + s[:, 128:]
    s = s[:, :64] + s[:, 64:]
    out_ref[...] = s[:, :32] + s[:, 32:]


def _elu(v):
    return jnp.where(v > 0, v, jnp.exp(jnp.minimum(v, 0.0)) - 1.0)


def _final_body(agg2_ref, x_ref, n2s_ref, s2g_ref, root_w_ref, conv_b_ref,
                fc1_w_ref, fc1_b_ref, fc2_w_ref, fc2_b_ref,
                fc3_w_ref, fc3_b_ref, out_ref, xch_ref, xcl_ref):
    x = x_ref[...]
    x16 = x[:, :D_INT]
    agg = agg2_ref[0] + agg2_ref[1]
    x_h = _elu(agg + jnp.dot(x16.astype(jnp.bfloat16),
                             root_w_ref[...].astype(jnp.bfloat16),
                             preferred_element_type=jnp.float32)
               + conv_b_ref[...])
    ones = jnp.ones((N, 1), jnp.float32)
    zeros = jnp.zeros((N, 15), jnp.float32)
    xc = jnp.concatenate([x_h, x[:, D_INT:], ones, zeros], axis=1)
    xc_hi = xc.astype(jnp.bfloat16)
    xc_lo = (xc - xc_hi.astype(jnp.float32)).astype(jnp.bfloat16)
    zpad = jnp.zeros((NPAD - N, 64), jnp.bfloat16)
    xch_ref[...] = jnp.concatenate([xc_hi, zpad], axis=0)
    xcl_ref[...] = jnp.concatenate([xc_lo, zpad], axis=0)

    SB = 512
    CHN = 1024

    def seg_block(sb):
        def body(ci, acc):
            seg_row = n2s_ref[0:1, pl.ds(ci * CHN, CHN)]
            ids = (lax.broadcasted_iota(jnp.int32, (SB, CHN), 0)
                   + (sb * SB)).astype(jnp.float32)
            oh = (ids == seg_row).astype(jnp.bfloat16)
            hi = xch_ref[pl.ds(ci * CHN, CHN), :]
            lo = xcl_ref[pl.ds(ci * CHN, CHN), :]
            return (acc
                    + jnp.dot(oh, hi, preferred_element_type=jnp.float32)
                    + jnp.dot(oh, lo, preferred_element_type=jnp.float32))
        return lax.fori_loop(0, NPAD // CHN, body,
                             jnp.zeros((SB, 64), jnp.float32))

    s1 = jnp.concatenate([seg_block(sb) for sb in range(NSUBP // SB)], axis=0)
    cnt1 = jnp.maximum(s1[:, 48:49], 1.0)
    xs = s1[:, :48] / cnt1

    s2g = s2g_ref[...]
    ids2 = lax.broadcasted_iota(jnp.int32, (NGRAPH, NSUBP), 0).astype(jnp.float32)
    oh2 = (ids2 == s2g[0:1, :]).astype(jnp.bfloat16)
    xs2 = jnp.concatenate([xs, jnp.ones((NSUBP, 1), jnp.float32),
                           jnp.zeros((NSUBP, 15), jnp.float32)], axis=1)
    xs2_hi = xs2.astype(jnp.bfloat16)
    xs2_lo = (xs2 - xs2_hi.astype(jnp.float32)).astype(jnp.bfloat16)
    s2 = (jnp.dot(oh2, xs2_hi, preferred_element_type=jnp.float32)
          + jnp.dot(oh2, xs2_lo, preferred_element_type=jnp.float32))
    cnt2 = jnp.maximum(s2[:, 48:49], 1.0)
    xg = s2[:, :48] / cnt2

    o = _elu(jnp.dot(xg.astype(jnp.bfloat16),
                     fc1_w_ref[...].astype(jnp.bfloat16),
                     preferred_element_type=jnp.float32) + fc1_b_ref[...])
    o = _elu(jnp.dot(o.astype(jnp.bfloat16),
                     fc2_w_ref[...].astype(jnp.bfloat16),
                     preferred_element_type=jnp.float32) + fc2_b_ref[...])
    o = jnp.dot(o.astype(jnp.bfloat16), fc3_w_ref[...].astype(jnp.bfloat16),
                preferred_element_type=jnp.float32) + fc3_b_ref[...]
    out_ref[...] = o


def kernel(x, edge_index, edge_attr, node_to_subgraph, subgraph_to_graph,
           nn_w1, nn_b1, nn_w2, nn_b2, root_w, conv_b,
           fc1_w, fc1_b, fc2_w, fc2_b, fc3_w, fc3_b):
    src = jnp.pad(edge_index[0], (0, EP - E))
    dst = jnp.pad(edge_index[1], (0, EP - E), constant_values=N)
    x16 = x[:, :D_INT]

    xg = _gather_kernel()(src, x16.astype(jnp.bfloat16))

    ea_t = jnp.pad(edge_attr.T, ((0, 0), (0, EP - E)))
    r_oh = (jnp.arange(D_INT * M_OUT)[None, :] // M_OUT
            == jnp.arange(D_INT)[:, None]).astype(jnp.bfloat16)
    msg = pl.pallas_call(
        _msg_body,
        grid=(EP // ET,),
        in_specs=[
            pl.BlockSpec((D_EDGE, ET), lambda i: (0, i)),
            pl.BlockSpec((ET, D_INT), lambda i: (i, 0)),
            pl.BlockSpec((D_EDGE, HID), lambda i: (0, 0)),
            pl.BlockSpec((1, HID), lambda i: (0, 0)),
            pl.BlockSpec((HID, D_INT * M_OUT), lambda i: (0, 0)),
            pl.BlockSpec((1, D_INT * M_OUT), lambda i: (0, 0)),
            pl.BlockSpec((D_INT, D_INT * M_OUT), lambda i: (0, 0)),
        ],
        out_specs=pl.BlockSpec((ET, M_OUT), lambda i: (i, 0)),
        out_shape=jax.ShapeDtypeStruct((EP, M_OUT), jnp.float32),
    )(ea_t, xg, nn_w1, nn_b1.reshape(1, HID),
      nn_w2, nn_b2.reshape(1, D_INT * M_OUT), r_oh)

    agg2 = _scatter_kernel()(dst, msg, jnp.zeros((NACC, M_OUT), jnp.float32))

    n2s_f = jnp.pad(node_to_subgraph, (0, NPAD - N),
                    constant_values=-1).astype(jnp.float32).reshape(1, NPAD)
    s2g_f = jnp.pad(subgraph_to_graph, (0, NSUBP - NSUB),
                    constant_values=-1).astype(jnp.float32).reshape(1, NSUBP)
    out = pl.pallas_call(
        _final_body,
        out_shape=jax.ShapeDtypeStruct((NGRAPH, 1), jnp.float32),
        scratch_shapes=[pltpu.VMEM((NPAD, 64), jnp.bfloat16),
                        pltpu.VMEM((NPAD, 64), jnp.bfloat16)],
    )(agg2, x, n2s_f, s2g_f, root_w, conv_b.reshape(1, M_OUT),
      fc1_w, fc1_b.reshape(1, 24), fc2_w, fc2_b.reshape(1, 12),
      fc3_w, fc3_b.reshape(1, 1))
    return out.reshape(-1)


# speedup vs baseline: 4.0720x; 1.0745x over previous
import functools

import jax
import jax.numpy as jnp
from jax import lax
from jax.experimental import pallas as pl
from jax.experimental.pallas import tpu as pltpu
from jax.experimental.pallas import tpu_sc as plsc

N = 10000
NPAD = 10240
E = 160000
EP = 163840
ET = 2048
NSUB = 2000
NSUBP = 2048
NGRAPH = 64
D_INT = 16
D_EDGE = 4
M_OUT = 32
HID = 128

NC = 2
NS = 16
NW = NC * NS
EH = EP // 2
EHW = EH // NW
CHG = EHW
CHS = EHW
ROWS_PER_TILE = N // NS
NACC = N + 8

@functools.cache
def _gather_kernel():
    mesh = plsc.VectorSubcoreMesh(core_axis_name="c", subcore_axis_name="s")

    @functools.partial(
        pl.kernel,
        mesh=mesh,
        out_type=jax.ShapeDtypeStruct((EH, D_INT), jnp.bfloat16),
        compiler_params=pltpu.CompilerParams(use_tc_tiling_on_sc=False),
        scratch_types=[
            pltpu.VMEM((CHG,), jnp.int32),
            pltpu.VMEM((CHG, D_INT), jnp.bfloat16),
            pltpu.SemaphoreType.DMA,
        ],
    )
    def _gather_k(src_hbm, x16_hbm, out_hbm, idx_v, rows_v, sem):
        wid = lax.axis_index("s") * NC + lax.axis_index("c")
        base = wid * EHW
        pltpu.sync_copy(src_hbm.at[pl.ds(base, CHG)], idx_v)
        pltpu.async_copy(x16_hbm.at[idx_v], rows_v, sem).wait()
        pltpu.sync_copy(rows_v, out_hbm.at[pl.ds(base, CHG)])

    return _gather_k


@functools.cache
def _scatter_kernel():
    mesh = plsc.VectorSubcoreMesh(core_axis_name="c", subcore_axis_name="s")

    @functools.partial(
        pl.kernel,
        mesh=mesh,
        out_type=jax.ShapeDtypeStruct((NC, N, M_OUT), jnp.float32),
        compiler_params=pltpu.CompilerParams(use_tc_tiling_on_sc=False),
        scratch_types=[
            pltpu.VMEM((CHS,), jnp.int32),
            pltpu.VMEM((CHS, M_OUT), jnp.float32),
            pltpu.VMEM_SHARED((NACC, M_OUT), jnp.float32),
            pltpu.SemaphoreType.DMA,
        ],
    )
    def _scatter_k(dst_hbm, msg_hbm, zeros_hbm, out_hbm, idx_v, rows_v,
                   acc_sh, sem):
        cid = lax.axis_index("c")
        sid = lax.axis_index("s")

        @pl.when(sid == 0)
        def _init():
            pltpu.sync_copy(zeros_hbm, acc_sh)

        plsc.subcore_barrier()

        off = cid * (EH // NC) + sid * (EH // NC // NS)
        pltpu.sync_copy(dst_hbm.at[pl.ds(off, CHS)], idx_v)
        pltpu.sync_copy(msg_hbm.at[pl.ds(off, CHS)], rows_v)
        pltpu.sync_copy(rows_v, acc_sh.at[idx_v], add=True)
        plsc.subcore_barrier()

        row0 = sid * ROWS_PER_TILE
        pltpu.sync_copy(acc_sh.at[pl.ds(row0, ROWS_PER_TILE)],
                        out_hbm.at[cid].at[pl.ds(row0, ROWS_PER_TILE)])

    return _scatter_k


def _msg_body(ea_ref, xg_ref, w1_ref, b1_ref, w2_ref, b2_ref, r_ref, out_ref):
    h = jnp.maximum(
        lax.dot_general(ea_ref[...].astype(jnp.bfloat16),
                        w1_ref[...].astype(jnp.bfloat16),
                        (((0,), (0,)), ((), ())),
                        preferred_element_type=jnp.float32)
        + b1_ref[...], 0.0)
    w = jnp.dot(h.astype(jnp.bfloat16), w2_ref[...].astype(jnp.bfloat16),
                preferred_element_type=jnp.float32) + b2_ref[...]
    w = w.astype(jnp.bfloat16).astype(jnp.float32)
    xg_e = jnp.dot(xg_ref[...], r_ref[...],
                   preferred_element_type=jnp.float32)
    s = xg_e * w
    s = s[:, :256] + s[:, 256:]
    s = s[:, :128] + s[:, 128:]
    s = s[:, :64] + s[:, 64:]
    out_ref[...] = s[:, :32] + s[:, 32:]


def _elu(v):
    return jnp.where(v > 0, v, jnp.exp(jnp.minimum(v, 0.0)) - 1.0)


def _final_body(agg2a_ref, agg2b_ref, x_ref, n2s_ref, s2g_ref, root_w_ref,
                conv_b_ref, fc1_w_ref, fc1_b_ref, fc2_w_ref, fc2_b_ref,
                fc3_w_ref, fc3_b_ref, out_ref, xch_ref, xcl_ref):
    x = x_ref[...]
    x16 = x[:, :D_INT]
    agg = ((agg2a_ref[0] + agg2a_ref[1])
           + (agg2b_ref[0] + agg2b_ref[1]))
    x_h = _elu(agg + jnp.dot(x16.astype(jnp.bfloat16),
                             root_w_ref[...].astype(jnp.bfloat16),
                             preferred_element_type=jnp.float32)
               + conv_b_ref[...])
    ones = jnp.ones((N, 1), jnp.float32)
    zeros = jnp.zeros((N, 15), jnp.float32)
    xc = jnp.concatenate([x_h, x[:, D_INT:], ones, zeros], axis=1)
    xc_hi = xc.astype(jnp.bfloat16)
    xc_lo = (xc - xc_hi.astype(jnp.float32)).astype(jnp.bfloat16)
    zpad = jnp.zeros((NPAD - N, 64), jnp.bfloat16)
    xch_ref[...] = jnp.concatenate([xc_hi, zpad], axis=0)
    xcl_ref[...] = jnp.concatenate([xc_lo, zpad], axis=0)

    SB = 512
    CHN = 1024

    def seg_block(sb):
        def body(ci, acc):
            seg_row = n2s_ref[0:1, pl.ds(ci * CHN, CHN)]
            ids = (lax.broadcasted_iota(jnp.int32, (SB, CHN), 0)
                   + (sb * SB)).astype(jnp.float32)
            oh = (ids == seg_row).astype(jnp.bfloat16)
            hi = xch_ref[pl.ds(ci * CHN, CHN), :]
            lo = xcl_ref[pl.ds(ci * CHN, CHN), :]
            return (acc
                    + jnp.dot(oh, hi, preferred_element_type=jnp.float32)
                    + jnp.dot(oh, lo, preferred_element_type=jnp.float32))
        return lax.fori_loop(0, NPAD // CHN, body,
                             jnp.zeros((SB, 64), jnp.float32))

    s1 = jnp.concatenate([seg_block(sb) for sb in range(NSUBP // SB)], axis=0)
    cnt1 = jnp.maximum(s1[:, 48:49], 1.0)
    xs = s1[:, :48] / cnt1

    s2g = s2g_ref[...]
    ids2 = lax.broadcasted_iota(jnp.int32, (NGRAPH, NSUBP), 0).astype(jnp.float32)
    oh2 = (ids2 == s2g[0:1, :]).astype(jnp.bfloat16)
    xs2 = jnp.concatenate([xs, jnp.ones((NSUBP, 1), jnp.float32),
                           jnp.zeros((NSUBP, 15), jnp.float32)], axis=1)
    xs2_hi = xs2.astype(jnp.bfloat16)
    xs2_lo = (xs2 - xs2_hi.astype(jnp.float32)).astype(jnp.bfloat16)
    s2 = (jnp.dot(oh2, xs2_hi, preferred_element_type=jnp.float32)
          + jnp.dot(oh2, xs2_lo, preferred_element_type=jnp.float32))
    cnt2 = jnp.maximum(s2[:, 48:49], 1.0)
    xg = s2[:, :48] / cnt2

    o = _elu(jnp.dot(xg.astype(jnp.bfloat16),
                     fc1_w_ref[...].astype(jnp.bfloat16),
                     preferred_element_type=jnp.float32) + fc1_b_ref[...])
    o = _elu(jnp.dot(o.astype(jnp.bfloat16),
                     fc2_w_ref[...].astype(jnp.bfloat16),
                     preferred_element_type=jnp.float32) + fc2_b_ref[...])
    o = jnp.dot(o.astype(jnp.bfloat16), fc3_w_ref[...].astype(jnp.bfloat16),
                preferred_element_type=jnp.float32) + fc3_b_ref[...]
    out_ref[...] = o


def kernel(x, edge_index, edge_attr, node_to_subgraph, subgraph_to_graph,
           nn_w1, nn_b1, nn_w2, nn_b2, root_w, conv_b,
           fc1_w, fc1_b, fc2_w, fc2_b, fc3_w, fc3_b):
    src = jnp.pad(edge_index[0], (0, EP - E))
    dst = jnp.pad(edge_index[1], (0, EP - E), constant_values=N)
    x16_bf = x[:, :D_INT].astype(jnp.bfloat16)

    ea_t = jnp.pad(edge_attr.T, ((0, 0), (0, EP - E)))
    r_oh = (jnp.arange(D_INT * M_OUT)[None, :] // M_OUT
            == jnp.arange(D_INT)[:, None]).astype(jnp.bfloat16)
    zeros_acc = jnp.zeros((NACC, M_OUT), jnp.float32)

    def msg_half(h, xg_h):
        return pl.pallas_call(
            _msg_body,
            grid=(EH // ET,),
            in_specs=[
                pl.BlockSpec((D_EDGE, ET), lambda i: (0, i + h * (EH // ET))),
                pl.BlockSpec((ET, D_INT), lambda i: (i, 0)),
                pl.BlockSpec((D_EDGE, HID), lambda i: (0, 0)),
                pl.BlockSpec((1, HID), lambda i: (0, 0)),
                pl.BlockSpec((HID, D_INT * M_OUT), lambda i: (0, 0)),
                pl.BlockSpec((1, D_INT * M_OUT), lambda i: (0, 0)),
                pl.BlockSpec((D_INT, D_INT * M_OUT), lambda i: (0, 0)),
            ],
            out_specs=pl.BlockSpec((ET, M_OUT), lambda i: (i, 0)),
            out_shape=jax.ShapeDtypeStruct((EH, M_OUT), jnp.float32),
        )(ea_t, xg_h, nn_w1, nn_b1.reshape(1, HID),
          nn_w2, nn_b2.reshape(1, D_INT * M_OUT), r_oh)

    xg0 = _gather_kernel()(src[:EH], x16_bf)
    xg1 = _gather_kernel()(src[EH:], x16_bf)
    msg0 = msg_half(0, xg0)
    msg1 = msg_half(1, xg1)
    agg2a = _scatter_kernel()(dst[:EH], msg0, zeros_acc)
    agg2b = _scatter_kernel()(dst[EH:], msg1, zeros_acc)

    n2s_f = jnp.pad(node_to_subgraph, (0, NPAD - N),
                    constant_values=-1).astype(jnp.float32).reshape(1, NPAD)
    s2g_f = jnp.pad(subgraph_to_graph, (0, NSUBP - NSUB),
                    constant_values=-1).astype(jnp.float32).reshape(1, NSUBP)
    out = pl.pallas_call(
        _final_body,
        out_shape=jax.ShapeDtypeStruct((NGRAPH, 1), jnp.float32),
        scratch_shapes=[pltpu.VMEM((NPAD, 64), jnp.bfloat16),
                        pltpu.VMEM((NPAD, 64), jnp.bfloat16)],
    )(agg2a, agg2b, x, n2s_f, s2g_f, root_w, conv_b.reshape(1, M_OUT),
      fc1_w, fc1_b.reshape(1, 24), fc2_w, fc2_b.reshape(1, 12),
      fc3_w, fc3_b.reshape(1, 1))
    return out.reshape(-1)
